# baseline (speedup 1.0000x reference)
import jax
import jax.numpy as jnp
from jax import lax
from jax.experimental import pallas as pl
from jax.experimental.pallas import tpu as pltpu
from jax.experimental.pallas import tpu_sc as plsc

N = 16384
C = 1000
L = 16
NS = 16
SPW = N // NS
G = 128
NG = SPW // G


def _nll_body(lp_hbm, tgt_hbm, out_hbm, refs):
    (tgt_v, idx_bufs, got_bufs, my_v, sem) = refs
    sid = lax.axis_index("s")
    base = sid * SPW

    pltpu.sync_copy(tgt_hbm.at[pl.ds(base, SPW)], tgt_v)

    lanes = lax.iota(jnp.int32, L)
    for g in range(NG):
        def idx_step(k, _, g=g):
            off = g * G + k * L
            t = tgt_v[pl.ds(off, L)]
            n = lanes + (base + off)
            idx_bufs[g][pl.ds(k * L, L)] = (
                ((t >> 3) * 128 + (n >> 7)) * 1024 + (t & 7) * 128
                + (n & 127))
            return 0
        lax.fori_loop(0, G // L, idx_step, 0)
        pltpu.async_copy(lp_hbm.at[idx_bufs[g]], got_bufs[g], sem)

    acc = jnp.zeros((L,), jnp.float32)
    for g in range(NG):
        pltpu.make_async_copy(
            lp_hbm.at[idx_bufs[g]], got_bufs[g], sem).wait()

        def sum_step(k, a, g=g):
            return a + got_bufs[g][pl.ds(k * L, L)]
        acc = lax.fori_loop(0, G // L, sum_step, acc)
    my_v[...] = acc * (-1.0 / N)
    pltpu.sync_copy(my_v, out_hbm.at[sid])


@jax.jit
def _nll_sc(lp_lin, tgt):
    mesh = plsc.VectorSubcoreMesh(
        core_axis_name="c", subcore_axis_name="s", num_cores=1)
    run = pl.kernel(
        _nll_body,
        mesh=mesh,
        out_type=jax.ShapeDtypeStruct((NS, L), jnp.float32),
        scratch_types=[(
            pltpu.VMEM((SPW,), jnp.int32),
            tuple(pltpu.VMEM((G,), jnp.int32) for _ in range(NG)),
            tuple(pltpu.VMEM((G,), jnp.float32) for _ in range(NG)),
            pltpu.VMEM((L,), jnp.float32),
            pltpu.SemaphoreType.DMA,
        )],
        compiler_params=pltpu.CompilerParams(needs_layout_passes=False),
    )
    return run(lp_lin, tgt)


def kernel(logprob, target):
    lp_lin = (logprob.T.reshape(C // 8, 8, N // 128, 128)
              .transpose(0, 2, 1, 3).reshape(-1))
    tgt = target.astype(jnp.int32)
    return jnp.sum(_nll_sc(lp_lin, tgt))


# speedup vs baseline: 1.0450x; 1.0450x over previous
import jax
import jax.numpy as jnp
from jax import lax
from jax.experimental import pallas as pl
from jax.experimental.pallas import tpu as pltpu
from jax.experimental.pallas import tpu_sc as plsc

N = 16384
C = 1000
L = 16
NS = 16
SPW = N // NS
G = 128
NG = SPW // G


def _nll_body(lp_hbm, tgt_hbm, out_hbm, refs):
    (tgt_v, idx_bufs, got_bufs, my_v, rb_v, row0_v, comb_v, shared,
     sem) = refs
    sid = lax.axis_index("s")
    base = sid * SPW

    pltpu.sync_copy(tgt_hbm.at[pl.ds(base, SPW)], tgt_v)

    lanes = lax.iota(jnp.int32, L)
    for g in range(NG):
        def idx_step(k, _, g=g):
            off = g * G + k * L
            t = tgt_v[pl.ds(off, L)]
            n = lanes + (base + off)
            idx_bufs[g][pl.ds(k * L, L)] = (
                ((t >> 3) * 128 + (n >> 7)) * 1024 + (t & 7) * 128
                + (n & 127))
            return 0
        lax.fori_loop(0, G // L, idx_step, 0)
        pltpu.async_copy(lp_hbm.at[idx_bufs[g]], got_bufs[g], sem)

    acc = jnp.zeros((L,), jnp.float32)
    for g in range(NG):
        pltpu.make_async_copy(
            lp_hbm.at[idx_bufs[g]], got_bufs[g], sem).wait()

        def sum_step(k, a, g=g):
            return a + got_bufs[g][pl.ds(k * L, L)]
        acc = lax.fori_loop(0, G // L, sum_step, acc)
    my_v[...] = acc

    @pl.when(sid == 0)
    def _():
        rb_v[...] = jnp.zeros((L,), jnp.float32)
        pltpu.sync_copy(rb_v, shared.at[0])
    plsc.subcore_barrier()

    row0_v[...] = lax.iota(jnp.int32, L)
    comb_v[0, :] = my_v[...]
    for i in range(1, NS):
        comb_v[i, :] = jnp.zeros((L,), jnp.float32)
    pltpu.sync_copy(comb_v, shared.at[row0_v], add=True)
    plsc.subcore_barrier()

    @pl.when(sid == 0)
    def _():
        pltpu.sync_copy(shared.at[0], rb_v)
        loss = jnp.sum(rb_v[...]) * (-1.0 / N)
        my_v[...] = jnp.full((L,), loss, jnp.float32)
        pltpu.sync_copy(my_v, out_hbm)


@jax.jit
def _nll_sc(lp_lin, tgt):
    mesh = plsc.VectorSubcoreMesh(
        core_axis_name="c", subcore_axis_name="s", num_cores=1)
    run = pl.kernel(
        _nll_body,
        mesh=mesh,
        out_type=jax.ShapeDtypeStruct((L,), jnp.float32),
        scratch_types=[(
            pltpu.VMEM((SPW,), jnp.int32),
            tuple(pltpu.VMEM((G,), jnp.int32) for _ in range(NG)),
            tuple(pltpu.VMEM((G,), jnp.float32) for _ in range(NG)),
            pltpu.VMEM((L,), jnp.float32),
            pltpu.VMEM((L,), jnp.float32),
            pltpu.VMEM((L,), jnp.int32),
            pltpu.VMEM((NS, L), jnp.float32),
            pltpu.VMEM_SHARED((NS, L), jnp.float32),
            pltpu.SemaphoreType.DMA,
        )],
        compiler_params=pltpu.CompilerParams(needs_layout_passes=False),
    )
    return run(lp_lin, tgt)


def kernel(logprob, target):
    lp_lin = (logprob.T.reshape(C // 8, 8, N // 128, 128)
              .transpose(0, 2, 1, 3).reshape(-1))
    tgt = target.astype(jnp.int32)
    return _nll_sc(lp_lin, tgt)[0]
